# Initial kernel scaffold; baseline (speedup 1.0000x reference)
import jax
import jax.numpy as jnp
from jax.experimental import pallas as pl


def kernel(inputs, W_ih, W_hh, b_ih, b_hh, W_out, b_out, W_root, W_pool, b_pool):
    raise NotImplementedError("write your pallas kernel here")


# speedup vs baseline: 7.6625x; 7.6625x over previous
import jax
import jax.numpy as jnp
from jax.experimental import pallas as pl
from jax.experimental.pallas import tpu as pltpu

N, T, D, H, KNN, C = 2048, 32, 128, 128, 16, 16
BR = 256
INV_DEG = 1.0 / float(KNN + 1)
M_EDGES = 16384.0
_DN_T = (((1,), (1,)), ((), ()))


def _lstm_body(x_ref, wih_ref, whh_ref, bih_ref, bhh_ref, out_ref, h_scr, c_scr,
               acc_scr):
    t = pl.program_id(0)

    @pl.when(t == 0)
    def _():
        h_scr[...] = jnp.zeros_like(h_scr)
        c_scr[...] = jnp.zeros_like(c_scr)

    x_t = x_ref[0]
    h_bf = h_scr[...].astype(jnp.bfloat16)
    gx = jax.lax.dot_general(x_t, wih_ref[...], _DN_T,
                             preferred_element_type=jnp.float32)
    acc_scr[...] = gx + bih_ref[...]
    acc_scr[...] += jax.lax.dot_general(h_bf, whh_ref[...], _DN_T,
                                        preferred_element_type=jnp.float32)
    gates = acc_scr[...] + bhh_ref[...]
    i = jax.nn.sigmoid(gates[:, 0:H])
    f = jax.nn.sigmoid(gates[:, H:2 * H])
    g = jnp.tanh(gates[:, 2 * H:3 * H])
    o = jax.nn.sigmoid(gates[:, 3 * H:4 * H])
    c = f * c_scr[...] + i * g
    h_new = o * jnp.tanh(c)
    c_scr[...] = c
    h_scr[...] = h_new

    @pl.when(t == T - 1)
    def _():
        out_ref[...] = h_new


def _lstm(xbf, wih_bf, whh_bf, bih, bhh):
    return pl.pallas_call(
        _lstm_body,
        grid=(T,),
        in_specs=[
            pl.BlockSpec((1, N, D), lambda t: (t, 0, 0)),
            pl.BlockSpec((4 * H, D), lambda t: (0, 0)),
            pl.BlockSpec((4 * H, H), lambda t: (0, 0)),
            pl.BlockSpec((1, 4 * H), lambda t: (0, 0)),
            pl.BlockSpec((1, 4 * H), lambda t: (0, 0)),
        ],
        out_specs=pl.BlockSpec((N, H), lambda t: (0, 0)),
        out_shape=jax.ShapeDtypeStruct((N, H), jnp.float32),
        scratch_shapes=[
            pltpu.VMEM((N, H), jnp.float32),
            pltpu.VMEM((N, H), jnp.float32),
            pltpu.VMEM((N, 4 * H), jnp.float32),
        ],
    )(xbf, wih_bf, whh_bf, bih, bhh)


def _knn_body(x_ref, sqr_ref, sqc_ref, nbr_ref, b_ref):
    r = pl.program_id(0)
    x = x_ref[...]
    xb = x_ref[pl.ds(r * BR, BR), :]
    sq_row = sqr_ref[...]
    sq_col = sqc_ref[pl.ds(r * BR, BR), :]
    g = jax.lax.dot_general(xb, x, _DN_T, preferred_element_type=jnp.float32)
    col = jax.lax.broadcasted_iota(jnp.int32, (BR, N), 1)
    rowid = r * BR + jax.lax.broadcasted_iota(jnp.int32, (BR, N), 0)
    d2 = (sq_col + sq_row) - g * 2.0
    d2 = d2 + jnp.where(col == rowid, jnp.float32(1e12), jnp.float32(0.0))

    kcol = jax.lax.broadcasted_iota(jnp.int32, (BR, KNN), 1)
    nbr0 = jnp.zeros((BR, KNN), jnp.int32)
    b0 = jnp.zeros((BR, N), jnp.float32)

    def body(k, carry):
        d2, bacc, nbracc = carry
        m = jnp.min(d2, axis=1, keepdims=True)
        cand = jnp.where(d2 <= m, col, N)
        jsel = jnp.min(cand, axis=1, keepdims=True)
        onehot = col == jsel
        d2 = jnp.where(onehot, jnp.float32(1e30), d2)
        bacc = jnp.where(onehot, jnp.float32(1.0), bacc)
        nbracc = jnp.where(kcol == k, jsel, nbracc)
        return d2, bacc, nbracc

    _, bacc, nbracc = jax.lax.fori_loop(0, KNN, body, (d2, b0, nbr0))
    nbr_ref[...] = nbracc
    b_ref[...] = bacc


def _knn(x, sqr, sqc):
    return pl.pallas_call(
        _knn_body,
        grid=(N // BR,),
        in_specs=[
            pl.BlockSpec((N, H), lambda r: (0, 0)),
            pl.BlockSpec((1, N), lambda r: (0, 0)),
            pl.BlockSpec((N, 1), lambda r: (0, 0)),
        ],
        out_specs=[
            pl.BlockSpec((BR, KNN), lambda r: (r, 0)),
            pl.BlockSpec((BR, N), lambda r: (r, 0)),
        ],
        out_shape=[
            jax.ShapeDtypeStruct((N, KNN), jnp.int32),
            jax.ShapeDtypeStruct((N, N), jnp.float32),
        ],
    )(x, sqr, sqc)


def _bx_body(b_ref, x_ref, out_ref):
    out_ref[...] = jnp.dot(b_ref[...], x_ref[...],
                           precision=jax.lax.Precision.HIGHEST,
                           preferred_element_type=jnp.float32)


def _bx(b, x):
    return pl.pallas_call(
        _bx_body,
        grid=(N // BR,),
        in_specs=[
            pl.BlockSpec((BR, N), lambda r: (r, 0)),
            pl.BlockSpec((N, H), lambda r: (0, 0)),
        ],
        out_specs=pl.BlockSpec((BR, H), lambda r: (r, 0)),
        out_shape=jax.ShapeDtypeStruct((N, H), jnp.float32),
    )(b, x)


def _gcn_body(x_ref, bx_ref, wout_ref, bout_ref, wroot_ref, wpool_ref,
              bpool_ref, s_ref):
    x = x_ref[...]
    agg = (bx_ref[...] + x) * INV_DEG
    x2 = jnp.dot(agg, wout_ref[...], preferred_element_type=jnp.float32)
    x2 = (x2 + bout_ref[...]) + jnp.dot(
        x, wroot_ref[...], preferred_element_type=jnp.float32)
    x2 = jax.nn.relu(x2)
    logits = jnp.dot(x2, wpool_ref[...], preferred_element_type=jnp.float32)
    logits = logits + bpool_ref[...]
    mx = jnp.max(logits, axis=1, keepdims=True)
    e = jnp.exp(logits - mx)
    s_ref[...] = e / jnp.sum(e, axis=1, keepdims=True)


def _gcn_softmax(x, bx, w_out, b_out, w_root, w_pool, b_pool):
    return pl.pallas_call(
        _gcn_body,
        grid=(1,),
        in_specs=[
            pl.BlockSpec((N, H), lambda i: (0, 0)),
            pl.BlockSpec((N, H), lambda i: (0, 0)),
            pl.BlockSpec((H, H), lambda i: (0, 0)),
            pl.BlockSpec((1, H), lambda i: (0, 0)),
            pl.BlockSpec((H, H), lambda i: (0, 0)),
            pl.BlockSpec((H, C), lambda i: (0, 0)),
            pl.BlockSpec((1, C), lambda i: (0, 0)),
        ],
        out_specs=pl.BlockSpec((N, C), lambda i: (0, 0)),
        out_shape=jax.ShapeDtypeStruct((N, C), jnp.float32),
    )(x, bx, w_out, b_out, w_root, w_pool, b_pool)


def _pool_body(s_ref, b_ref, spec_ref, orth_ref, clus_ref):
    s = s_ref[...]
    st = s.T
    b = b_ref[...]
    st_bf = st.astype(jnp.bfloat16)
    s_bf = s.astype(jnp.bfloat16)
    b_bf = b.astype(jnp.bfloat16)
    m1 = jax.lax.dot_general(st_bf, b_bf, _DN_T,
                             preferred_element_type=jnp.float32)
    m1b = m1.astype(jnp.bfloat16)
    out_adj = jax.lax.dot_general(m1b, s_bf, (((1,), (0,)), ((), ())),
                                  preferred_element_type=jnp.float32)
    degrees = jnp.sum(b, axis=0, keepdims=True)
    ca = jnp.sum(st_bf.astype(jnp.float32)
                 * degrees.astype(jnp.bfloat16).astype(jnp.float32),
                 axis=1, keepdims=True)
    ir = jax.lax.broadcasted_iota(jnp.int32, (C, C), 0)
    ic = jax.lax.broadcasted_iota(jnp.int32, (C, C), 1)
    eye = ir == ic
    normalizer = ((ca * ca.T) * 0.5) / M_EDGES
    dec = out_adj - normalizer
    tr = jnp.sum(jnp.where(eye, dec, jnp.float32(0.0)))
    spec_ref[...] = jnp.full((1, 1), (-tr) / 2.0 / M_EDGES)

    ss = jax.lax.dot_general(st, s, (((1,), (0,)), ((), ())),
                             preferred_element_type=jnp.float32)
    ssn = jnp.sqrt(jnp.sum(ss * ss))
    om = ss / ssn - jnp.where(eye, jnp.float32(0.25), jnp.float32(0.0))
    orth_ref[...] = jnp.full((1, 1), jnp.sqrt(jnp.sum(om * om)))

    cs = jnp.sum(s, axis=0, keepdims=True)
    clus = jnp.sqrt(jnp.sum(cs * cs)) / N * 4.0 - 1.0
    clus_ref[...] = jnp.full((1, 1), clus)


def _pool(s, b):
    return pl.pallas_call(
        _pool_body,
        grid=(1,),
        in_specs=[
            pl.BlockSpec((N, C), lambda i: (0, 0)),
            pl.BlockSpec((N, N), lambda i: (0, 0)),
        ],
        out_specs=[
            pl.BlockSpec((1, 1), lambda i: (0, 0)),
            pl.BlockSpec((1, 1), lambda i: (0, 0)),
            pl.BlockSpec((1, 1), lambda i: (0, 0)),
        ],
        out_shape=[
            jax.ShapeDtypeStruct((1, 1), jnp.float32),
            jax.ShapeDtypeStruct((1, 1), jnp.float32),
            jax.ShapeDtypeStruct((1, 1), jnp.float32),
        ],
    )(s, b)


def kernel(inputs, W_ih, W_hh, b_ih, b_hh, W_out, b_out, W_root, W_pool, b_pool):
    xbf = jnp.swapaxes(inputs, 0, 1).astype(jnp.bfloat16)
    x = _lstm(xbf, W_ih.astype(jnp.bfloat16), W_hh.astype(jnp.bfloat16),
              b_ih[None], b_hh[None])

    sq = jnp.sum(x * x, axis=-1)
    nbr, bmat = _knn(x, sq[None, :], sq[:, None])
    del nbr

    bx = _bx(bmat, x)
    s = _gcn_softmax(x, bx, W_out, b_out[None, :], W_root, W_pool,
                     b_pool[None, :])
    spec, orth, clus = _pool(s, bmat)
    return s[None], spec[0, 0], orth[0, 0], clus[0, 0]
